# Initial kernel scaffold; baseline (speedup 1.0000x reference)
import jax
import jax.numpy as jnp
from jax.experimental import pallas as pl


def kernel(binary_input, embeddings):
    raise NotImplementedError("write your pallas kernel here")


# speedup vs baseline: 16.7483x; 16.7483x over previous
import jax
import jax.numpy as jnp
from jax.experimental import pallas as pl

_SEQ_BLK = 256


def _body(bin_ref, emb_ref, out_ref, logit_ref):
    amp = bin_ref[...] * 2.0 - 1.0
    table = emb_ref[...]
    out_ref[...] = amp[:, :, None] * table[None, :, :]
    rowsum = jnp.sum(table, axis=1)
    logit_ref[...] = amp * rowsum[None, :]


def kernel(binary_input, embeddings):
    seq_len, blen = binary_input.shape
    vocab, emb_sz = embeddings.shape
    grid = (seq_len // _SEQ_BLK,)
    emb, logit = pl.pallas_call(
        _body,
        grid=grid,
        in_specs=[
            pl.BlockSpec((_SEQ_BLK, blen), lambda i: (i, 0)),
            pl.BlockSpec((vocab, emb_sz), lambda i: (0, 0)),
        ],
        out_specs=(
            pl.BlockSpec((_SEQ_BLK, blen, emb_sz), lambda i: (i, 0, 0)),
            pl.BlockSpec((_SEQ_BLK, blen), lambda i: (i, 0)),
        ),
        out_shape=(
            jax.ShapeDtypeStruct((seq_len, blen, emb_sz), jnp.float32),
            jax.ShapeDtypeStruct((seq_len, blen), jnp.float32),
        ),
    )(binary_input, embeddings)
    return emb, logit.reshape(seq_len, blen, 1)


# speedup vs baseline: 18.5731x; 1.1090x over previous
import jax
import jax.numpy as jnp
from jax.experimental import pallas as pl

_SEQ_BLK = 512


def _body(bin_ref, emb_ref, out_ref, logit_ref):
    amp = bin_ref[...] * 2.0 - 1.0
    table = emb_ref[...]
    out_ref[...] = amp[:, :, None] * table[None, :, :]
    rowsum = jnp.sum(table, axis=1)
    logit_ref[...] = amp * rowsum[None, :]


def kernel(binary_input, embeddings):
    seq_len, blen = binary_input.shape
    vocab, emb_sz = embeddings.shape
    grid = (seq_len // _SEQ_BLK,)
    emb, logit = pl.pallas_call(
        _body,
        grid=grid,
        in_specs=[
            pl.BlockSpec((_SEQ_BLK, blen), lambda i: (i, 0)),
            pl.BlockSpec((vocab, emb_sz), lambda i: (0, 0)),
        ],
        out_specs=(
            pl.BlockSpec((_SEQ_BLK, blen, emb_sz), lambda i: (i, 0, 0)),
            pl.BlockSpec((_SEQ_BLK, blen), lambda i: (i, 0)),
        ),
        out_shape=(
            jax.ShapeDtypeStruct((seq_len, blen, emb_sz), jnp.float32),
            jax.ShapeDtypeStruct((seq_len, blen), jnp.float32),
        ),
    )(binary_input, embeddings)
    return emb, logit.reshape(seq_len, blen, 1)
